# baseline (speedup 1.0000x reference)
import jax
import jax.numpy as jnp
from jax import lax
from jax.experimental import pallas as pl
from jax.experimental.pallas import tpu as pltpu
from jax.experimental.pallas import tpu_sc as plsc

_N = 10000
_E = 320000
_DIN = 128
_DH = 16
_DOUT = 128

_NC = 2
_NS = 16
_NW = _NC * _NS
_EPW = 10240
_EP = _NW * _EPW
_CR = 1024
_NCHUNK = _EPW // _CR
_DUMP = _N
_ACC_ROWS = 10240
_TBL_ROWS = _N + 16
_ZSL = _ACC_ROWS // _NS

_f32 = jnp.float32


def _deg_body(zeros_hbm, ones_hbm, src_hbm, dst_hbm, out_hbm, idxs_all,
              idxd_all, ones_v, acc_o, acc_i, sem_o, sem_i):
  cid = lax.axis_index("c")
  sid = lax.axis_index("s")
  wid = cid * _NS + sid

  pltpu.sync_copy(zeros_hbm.at[pl.ds(sid * _ZSL, _ZSL)],
                  acc_o.at[pl.ds(sid * _ZSL, _ZSL)])
  pltpu.sync_copy(zeros_hbm.at[pl.ds(sid * _ZSL, _ZSL)],
                  acc_i.at[pl.ds(sid * _ZSL, _ZSL)])
  pltpu.sync_copy(ones_hbm, ones_v)
  base = wid * _NCHUNK
  pltpu.sync_copy(src_hbm.at[pl.ds(base, _NCHUNK)], idxs_all)
  pltpu.sync_copy(dst_hbm.at[pl.ds(base, _NCHUNK)], idxd_all)
  plsc.subcore_barrier()

  def _chunk(c, carry):
    io = idxs_all.at[c]
    ii = idxd_all.at[c]
    pltpu.async_copy(ones_v, acc_o.at[io], sem_o, add=True)
    pltpu.async_copy(ones_v, acc_i.at[ii], sem_i, add=True)
    pltpu.make_async_copy(ones_v, acc_o.at[io], sem_o).wait()
    pltpu.make_async_copy(ones_v, acc_i.at[ii], sem_i).wait()
    return carry

  lax.fori_loop(0, _NCHUNK, _chunk, 0)
  plsc.subcore_barrier()

  pltpu.sync_copy(acc_o.at[pl.ds(sid * _ZSL, _ZSL)],
                  out_hbm.at[cid, 0, pl.ds(sid * _ZSL, _ZSL)])
  pltpu.sync_copy(acc_i.at[pl.ds(sid * _ZSL, _ZSL)],
                  out_hbm.at[cid, 1, pl.ds(sid * _ZSL, _ZSL)])


_deg_call = pl.kernel(
    _deg_body,
    out_type=jax.ShapeDtypeStruct((_NC, 2, _ACC_ROWS, _DH), _f32),
    mesh=plsc.VectorSubcoreMesh(core_axis_name="c", subcore_axis_name="s"),
    scratch_types=[
        pltpu.VMEM((_NCHUNK, _CR), jnp.int32),
        pltpu.VMEM((_NCHUNK, _CR), jnp.int32),
        pltpu.VMEM((_CR, _DH), _f32),
        pltpu.VMEM_SHARED((_ACC_ROWS, _DH), _f32),
        pltpu.VMEM_SHARED((_ACC_ROWS, _DH), _f32),
        pltpu.SemaphoreType.DMA,
        pltpu.SemaphoreType.DMA,
    ],
    compiler_params=pltpu.CompilerParams(use_tc_tiling_on_sc=False),
)


def _agg_body(zeros_hbm, h_hbm, src_hbm, dst_hbm, out_hbm, idxs_all, idxd_all,
              r_a, r_b, acc, sem_a, sem_b):
  cid = lax.axis_index("c")
  sid = lax.axis_index("s")
  wid = cid * _NS + sid

  pltpu.sync_copy(zeros_hbm.at[pl.ds(sid * _ZSL, _ZSL)],
                  acc.at[pl.ds(sid * _ZSL, _ZSL)])
  base = wid * _NCHUNK
  pltpu.sync_copy(src_hbm.at[pl.ds(base, _NCHUNK)], idxs_all)
  pltpu.sync_copy(dst_hbm.at[pl.ds(base, _NCHUNK)], idxd_all)
  plsc.subcore_barrier()

  def _fire(chunk, rows, sem):
    c = jnp.minimum(chunk, _NCHUNK - 1)
    pltpu.async_copy(h_hbm.at[idxs_all.at[c]], rows, sem)

  def _drain(rows, sem):
    pltpu.make_async_copy(h_hbm.at[pl.ds(0, _CR)], rows, sem).wait()

  def _scatter(chunk, rows):
    pltpu.sync_copy(rows, acc.at[idxd_all.at[chunk]], add=True)

  _fire(0, r_a, sem_a)

  def _pair(p, carry):
    ca = 2 * p
    cb = ca + 1
    _fire(cb, r_b, sem_b)
    _drain(r_a, sem_a)
    _scatter(ca, r_a)
    _fire(ca + 2, r_a, sem_a)
    _drain(r_b, sem_b)
    _scatter(cb, r_b)
    return carry

  lax.fori_loop(0, _NCHUNK // 2, _pair, 0)
  _drain(r_a, sem_a)
  plsc.subcore_barrier()

  pltpu.sync_copy(acc.at[pl.ds(sid * _ZSL, _ZSL)],
                  out_hbm.at[cid, pl.ds(sid * _ZSL, _ZSL)])


_agg_call = pl.kernel(
    _agg_body,
    out_type=jax.ShapeDtypeStruct((_NC, _ACC_ROWS, _DH), _f32),
    mesh=plsc.VectorSubcoreMesh(core_axis_name="c", subcore_axis_name="s"),
    scratch_types=[
        pltpu.VMEM((_NCHUNK, _CR), jnp.int32),
        pltpu.VMEM((_NCHUNK, _CR), jnp.int32),
        pltpu.VMEM((_CR, _DH), _f32),
        pltpu.VMEM((_CR, _DH), _f32),
        pltpu.VMEM_SHARED((_ACC_ROWS, _DH), _f32),
        pltpu.SemaphoreType.DMA,
        pltpu.SemaphoreType.DMA,
    ],
    compiler_params=pltpu.CompilerParams(use_tc_tiling_on_sc=False),
)


_GRID = 10
_BR = _N // _GRID


def _norms(deg_ref):
  deg_o = deg_ref[0, 0] + deg_ref[1, 0]
  deg_i = deg_ref[0, 1] + deg_ref[1, 1]
  ns = jnp.where(deg_o > 0, lax.rsqrt(jnp.maximum(deg_o, 1.0)), 1.0)
  nd = jnp.where(deg_i > 0, lax.rsqrt(jnp.maximum(deg_i, 1.0)), 1.0)
  return ns, nd


def _l1_body(x_ref, w_ref, deg_ref, o_ref):
  ns, _ = _norms(deg_ref)
  o_ref[...] = jnp.dot(x_ref[...], w_ref[...], preferred_element_type=_f32) * ns


_l1_call = pl.pallas_call(
    _l1_body,
    grid=(_GRID,),
    in_specs=[
        pl.BlockSpec((_BR, _DIN), lambda i: (i, 0)),
        pl.BlockSpec((_DIN, _DH), lambda i: (0, 0)),
        pl.BlockSpec((_NC, 2, _BR, _DH), lambda i: (0, 0, i, 0)),
    ],
    out_specs=pl.BlockSpec((_BR, _DH), lambda i: (i, 0)),
    out_shape=jax.ShapeDtypeStruct((_TBL_ROWS, _DH), _f32),
)


def _mid_body(a_ref, deg_ref, b_ref, o_ref):
  a = a_ref[0] + a_ref[1]
  ns, nd = _norms(deg_ref)
  h = jnp.maximum(a * nd + b_ref[...], 0.0)
  o_ref[...] = h * ns


_mid_call = pl.pallas_call(
    _mid_body,
    grid=(_GRID,),
    in_specs=[
        pl.BlockSpec((_NC, _BR, _DH), lambda i: (0, i, 0)),
        pl.BlockSpec((_NC, 2, _BR, _DH), lambda i: (0, 0, i, 0)),
        pl.BlockSpec((1, _DH), lambda i: (0, 0)),
    ],
    out_specs=pl.BlockSpec((_BR, _DH), lambda i: (i, 0)),
    out_shape=jax.ShapeDtypeStruct((_TBL_ROWS, _DH), _f32),
)


def _fin_body(a_ref, deg_ref, w_ref, b_ref, o_ref):
  a = a_ref[0] + a_ref[1]
  _, nd = _norms(deg_ref)
  o_ref[...] = (
      jnp.dot(a * nd, w_ref[...], preferred_element_type=_f32) + b_ref[...])


_fin_call = pl.pallas_call(
    _fin_body,
    grid=(_GRID,),
    in_specs=[
        pl.BlockSpec((_NC, _BR, _DH), lambda i: (0, i, 0)),
        pl.BlockSpec((_NC, 2, _BR, _DH), lambda i: (0, 0, i, 0)),
        pl.BlockSpec((_DH, _DOUT), lambda i: (0, 0)),
        pl.BlockSpec((1, _DOUT), lambda i: (0, 0)),
    ],
    out_specs=pl.BlockSpec((_BR, _DOUT), lambda i: (i, 0)),
    out_shape=jax.ShapeDtypeStruct((_N, _DOUT), _f32),
)


@jax.jit
def kernel(x, edge_index, W1, b1, W2, b2):
  src = edge_index[0].astype(jnp.int32)
  dst = edge_index[1].astype(jnp.int32)
  pad = _EP - _E
  padv = jnp.full((pad,), _DUMP, jnp.int32)
  srcp = jnp.concatenate([src, padv]).reshape(_EP // _CR, _CR)
  dstp = jnp.concatenate([dst, padv]).reshape(_EP // _CR, _CR)

  zeros_acc = jnp.zeros((_ACC_ROWS, _DH), _f32)
  ones_cr = jnp.ones((_CR, _DH), _f32)

  degp = _deg_call(zeros_acc, ones_cr, srcp, dstp)
  h1t = _l1_call(x, W1, degp)
  a1p = _agg_call(zeros_acc, h1t, srcp, dstp)
  h2t = _mid_call(a1p, degp, b1.reshape(1, _DH))
  a2p = _agg_call(zeros_acc, h2t, srcp, dstp)
  return _fin_call(a2p, degp, W2, b2.reshape(1, _DOUT))
